# baseline (speedup 1.0000x reference)
import functools

import jax
import jax.numpy as jnp
from jax import lax
from jax.experimental import pallas as pl
from jax.experimental.pallas import tpu as pltpu
from jax.experimental.pallas import tpu_sc as plsc

N_NODES = 10000
D = 128
NG = 64
NO = 10
NC = 2
NS = 16
NW = NC * NS
CH = 128

R = 1000
GRID = N_NODES // R
N_PAD = 10240
DEG_PAD = 10240


def _sc_degree(edge_ref_arr, E):
    nrows = E // CH
    IS = 8
    mesh = plsc.VectorSubcoreMesh(core_axis_name="c", subcore_axis_name="s")

    @functools.partial(
        pl.kernel,
        out_type=jax.ShapeDtypeStruct((NC, DEG_PAD), jnp.float32),
        mesh=mesh,
        scratch_types=[
            pltpu.VMEM((640,), jnp.float32),
            pltpu.VMEM((CH,), jnp.float32),
            pltpu.VMEM((IS, CH), jnp.int32),
            pltpu.VMEM_SHARED((DEG_PAD,), jnp.float32),
            pltpu.SemaphoreType.DMA((IS,)),
            pltpu.SemaphoreType.DMA((IS,)),
        ],
    )
    def k(e_ref, out_ref, zbuf, ones, didx, acc, jsem, ssem):
        c = lax.axis_index("c")
        s = lax.axis_index("s")
        wid = c * NS + s
        nch = (nrows - wid + NW - 1) // NW

        for p in range(2):
            dbase = pl.multiple_of((wid + p * NW) * CH, 8)
            pltpu.async_copy(e_ref.at[1, pl.ds(dbase, CH)], didx.at[p],
                             jsem.at[p])

        def zfill(i, carry):
            zbuf[pl.ds(i * 16, 16)] = jnp.zeros((16,), jnp.float32)
            return carry

        lax.fori_loop(0, 40, zfill, 0)
        for i in range(CH // 16):
            ones[pl.ds(i * 16, 16)] = jnp.ones((16,), jnp.float32)
        pltpu.sync_copy(zbuf, acc.at[pl.ds(s * 640, 640)])
        plsc.subcore_barrier()

        def body(j, carry):
            b = j % IS
            mb = pl.multiple_of((wid + j * NW) * CH, 8)
            pltpu.make_async_copy(e_ref.at[1, pl.ds(mb, CH)], didx.at[b],
                                  jsem.at[b]).wait()
            pltpu.async_copy(ones, acc.at[didx.at[b]], ssem.at[b], add=True)

            @pl.when(j + 2 < nch)
            def _():
                bn = (j + 2) % IS

                @pl.when(j >= 6)
                def _():
                    bo = (j - 6) % IS
                    pltpu.make_async_copy(ones, acc.at[didx.at[bo]],
                                          ssem.at[bo]).wait()

                mb2 = pl.multiple_of((wid + (j + 2) * NW) * CH, 8)
                pltpu.async_copy(e_ref.at[1, pl.ds(mb2, CH)], didx.at[bn],
                                 jsem.at[bn])

            return carry

        lax.fori_loop(0, nch, body, 0)
        for dj in range(8):
            jj = nch - 8 + dj

            @pl.when(jj >= 0)
            def _():
                b = jj % IS
                pltpu.make_async_copy(ones, acc.at[didx.at[b]],
                                      ssem.at[b]).wait()

        plsc.subcore_barrier()
        pltpu.sync_copy(acc.at[pl.ds(s * 640, 640)],
                        out_ref.at[c, pl.ds(s * 640, 640)])

    return k(edge_ref_arr)


def _sc_aggregate(g, edge_ref_arr, E):
    nrows = E // CH
    rpt = N_PAD // NS
    RS = 2
    IS = 6
    mesh = plsc.VectorSubcoreMesh(core_axis_name="c", subcore_axis_name="s")

    @functools.partial(
        pl.kernel,
        out_type=jax.ShapeDtypeStruct((NC, N_PAD, D), jnp.float32),
        mesh=mesh,
        scratch_types=[
            pltpu.VMEM((RS, CH, D), jnp.float32),
            pltpu.VMEM((IS, CH), jnp.int32),
            pltpu.VMEM((IS, CH), jnp.int32),
            pltpu.VMEM_SHARED((N_PAD, D), jnp.float32),
            pltpu.SemaphoreType.DMA((IS,)),
            pltpu.SemaphoreType.DMA((IS,)),
            pltpu.SemaphoreType.DMA((RS,)),
            pltpu.SemaphoreType.DMA((RS,)),
        ],
    )
    def k(g_ref, e_ref, out_ref, rows, sidx, didx, acc,
          isem, jsem, gsem, ssem):
        c = lax.axis_index("c")
        s = lax.axis_index("s")
        wid = c * NS + s
        nch = (nrows - wid + NW - 1) // NW

        for p in range(2):
            sb = pl.multiple_of((wid + p * NW) * CH, 8)
            pltpu.async_copy(e_ref.at[0, pl.ds(sb, CH)], sidx.at[p],
                             isem.at[p])
            pltpu.async_copy(e_ref.at[1, pl.ds(sb, CH)], didx.at[p],
                             jsem.at[p])

        def zrow(i, carry):
            for k16 in range(D // 16):
                rows[RS - 1, i, pl.ds(k16 * 16, 16)] = jnp.zeros(
                    (16,), jnp.float32)
            return carry

        lax.fori_loop(0, CH, zrow, 0)
        rbase = s * rpt
        for t in range(rpt // CH):
            pltpu.sync_copy(rows.at[RS - 1], acc.at[pl.ds(rbase + t * CH, CH)])
        sb0 = pl.multiple_of(wid * CH, 8)
        pltpu.make_async_copy(e_ref.at[0, pl.ds(sb0, CH)], sidx.at[0],
                              isem.at[0]).wait()
        pltpu.async_copy(g_ref.at[sidx.at[0]], rows.at[0], gsem.at[0])
        plsc.subcore_barrier()

        def body(j, carry):
            b = j % RS
            ib = j % IS

            @pl.when(j + 1 < nch)
            def _():
                bn = (j + 1) % RS
                ibn = (j + 1) % IS

                @pl.when(j + 1 - RS >= 0)
                def _():
                    jo = j + 1 - RS
                    pltpu.make_async_copy(rows.at[bn],
                                          acc.at[didx.at[jo % IS]],
                                          ssem.at[bn]).wait()

                sb1 = pl.multiple_of((wid + (j + 1) * NW) * CH, 8)
                pltpu.make_async_copy(e_ref.at[0, pl.ds(sb1, CH)],
                                      sidx.at[ibn], isem.at[ibn]).wait()
                pltpu.async_copy(g_ref.at[sidx.at[ibn]], rows.at[bn],
                                 gsem.at[bn])

            pltpu.make_async_copy(g_ref.at[sidx.at[ib]], rows.at[b],
                                  gsem.at[b]).wait()
            db0 = pl.multiple_of((wid + j * NW) * CH, 8)
            pltpu.make_async_copy(e_ref.at[1, pl.ds(db0, CH)], didx.at[ib],
                                  jsem.at[ib]).wait()
            pltpu.async_copy(rows.at[b], acc.at[didx.at[ib]], ssem.at[b],
                             add=True)

            @pl.when(j + 2 < nch)
            def _():
                ib2 = (j + 2) % IS
                sb2 = pl.multiple_of((wid + (j + 2) * NW) * CH, 8)
                pltpu.async_copy(e_ref.at[0, pl.ds(sb2, CH)], sidx.at[ib2],
                                 isem.at[ib2])
                pltpu.async_copy(e_ref.at[1, pl.ds(sb2, CH)], didx.at[ib2],
                                 jsem.at[ib2])

            return carry

        lax.fori_loop(0, nch, body, 0)
        for dj in range(RS):
            jj = nch - RS + dj

            @pl.when(jj >= 0)
            def _():
                pltpu.make_async_copy(rows.at[jj % RS],
                                      acc.at[didx.at[jj % IS]],
                                      ssem.at[jj % RS]).wait()

        plsc.subcore_barrier()
        pltpu.sync_copy(acc.at[pl.ds(rbase, rpt)],
                        out_ref.at[c, pl.ds(rbase, rpt)])

    return k(g, edge_ref_arr)


def _dinv_block(d_ref):
    deg = d_ref[...].astype(jnp.float32)
    return lax.rsqrt(jnp.maximum(deg, 1.0))


def _tc1_body(x_ref, w_ref, d_ref, g_ref):
    dinv = _dinv_block(d_ref)
    h = jnp.dot(x_ref[...].astype(jnp.bfloat16), w_ref[...],
                preferred_element_type=jnp.float32)
    g_ref[...] = h * dinv


def _tc2_body(a_ref0, a_ref1, g1_ref, d_ref, w_ref, b_ref, g2_ref):
    dinv = _dinv_block(d_ref)
    agg = (a_ref0[0] + a_ref1[0] + g1_ref[...]).astype(jnp.float32)
    x2 = jnp.maximum(dinv * agg + b_ref[...], 0.0)
    h = jnp.dot(x2.astype(jnp.bfloat16), w_ref[...],
                preferred_element_type=jnp.float32)
    g2_ref[...] = h * dinv


def _cnt_body(batch_ref, cnt_ref):
    i = pl.program_id(0)
    bb = batch_ref[0, 0, :]
    onehot = jnp.equal(
        jnp.reshape(bb, (R, 1)),
        lax.broadcasted_iota(jnp.int32, (R, NG), 1)).astype(jnp.float32)
    pc = lax.dot_general(onehot, jnp.ones((R, D), jnp.float32),
                         (((0,), (0,)), ((), ())),
                         preferred_element_type=jnp.float32)

    @pl.when(i == 0)
    def _():
        cnt_ref[...] = pc

    @pl.when(i > 0)
    def _():
        cnt_ref[...] += pc


def _tc3_body(a_ref0, a_ref1, g2_ref, d_ref, b_ref, batch_ref, cnt_ref,
              wc_ref, bc_ref, out_ref, sums):
    i = pl.program_id(0)
    dinv = _dinv_block(d_ref)
    agg = (a_ref0[0] + a_ref1[0] + g2_ref[...]).astype(jnp.float32)
    h3 = jnp.maximum(dinv * agg + b_ref[...], 0.0)
    bb = batch_ref[0, 0, :]
    onehot = jnp.equal(
        jnp.reshape(bb, (R, 1)),
        lax.broadcasted_iota(jnp.int32, (R, NG), 1)).astype(jnp.float32)
    ps = lax.dot_general(onehot.astype(jnp.bfloat16),
                         h3.astype(jnp.bfloat16), (((0,), (0,)), ((), ())),
                         preferred_element_type=jnp.float32)

    @pl.when(i == 0)
    def _():
        sums[...] = ps

    @pl.when(i > 0)
    def _():
        sums[...] += ps

    @pl.when(i == GRID - 1)
    def _():
        pooled = sums[...] / jnp.maximum(cnt_ref[...], 1.0)
        logits = jnp.dot(pooled, wc_ref[...],
                         preferred_element_type=jnp.float32) + bc_ref[...]
        m = jnp.max(logits, axis=1, keepdims=True)
        sh = logits - m
        lse = jnp.log(jnp.sum(jnp.exp(sh), axis=1, keepdims=True))
        out_ref[...] = (sh - lse)[:, :NO]


def kernel(x, edge_index, batch, W1, b1, W2, b2, Wc, bc):
    E = edge_index.shape[1]

    batch3 = jnp.reshape(batch, (GRID, 1, R))
    cnts = pl.pallas_call(
        _cnt_body,
        grid=(GRID,),
        in_specs=[pl.BlockSpec((1, 1, R), lambda i: (i, 0, 0))],
        out_specs=pl.BlockSpec((NG, D), lambda i: (0, 0)),
        out_shape=jax.ShapeDtypeStruct((NG, D), jnp.float32),
    )(batch3)

    deg = _sc_degree(edge_index, E)
    dsum = jnp.reshape((deg[0] + deg[1]).astype(jnp.bfloat16), (DEG_PAD, 1))

    row_spec = pl.BlockSpec((R, D), lambda i: (i, 0))
    aspec0 = pl.BlockSpec((1, R, D), lambda i: (0, i, 0))
    aspec1 = pl.BlockSpec((1, R, D), lambda i: (1, i, 0))
    dspec = pl.BlockSpec((R, 1), lambda i: (i, 0))
    wspec = pl.BlockSpec((D, D), lambda i: (0, 0))
    bspec = pl.BlockSpec((1, D), lambda i: (0, 0))

    g1 = pl.pallas_call(
        _tc1_body,
        grid=(GRID,),
        in_specs=[row_spec, wspec, dspec],
        out_specs=row_spec,
        out_shape=jax.ShapeDtypeStruct((N_NODES, D), jnp.float32),
    )(x, W1.astype(jnp.bfloat16), dsum)

    a1 = _sc_aggregate(g1, edge_index, E)

    g2 = pl.pallas_call(
        _tc2_body,
        grid=(GRID,),
        in_specs=[aspec0, aspec1, row_spec, dspec, wspec, bspec],
        out_specs=row_spec,
        out_shape=jax.ShapeDtypeStruct((N_NODES, D), jnp.float32),
    )(a1, a1, g1, dsum, W2.astype(jnp.bfloat16), jnp.reshape(b1, (1, D)))

    a2 = _sc_aggregate(g2, edge_index, E)

    wc_pad = jnp.zeros((D, D), jnp.float32).at[:, :Wc.shape[1]].set(Wc)
    bc_pad = jnp.full((1, D), -1e30, jnp.float32).at[0, :bc.shape[0]].set(bc)

    logits_pad = pl.pallas_call(
        _tc3_body,
        grid=(GRID,),
        in_specs=[aspec0, aspec1, row_spec, dspec, bspec,
                  pl.BlockSpec((1, 1, R), lambda i: (i, 0, 0)),
                  pl.BlockSpec((NG, D), lambda i: (0, 0)),
                  wspec, bspec],
        out_specs=pl.BlockSpec((NG, NO), lambda i: (0, 0)),
        out_shape=jax.ShapeDtypeStruct((NG, NO), jnp.float32),
        scratch_shapes=[pltpu.VMEM((NG, D), jnp.float32)],
    )(a2, a2, g2, dsum, jnp.reshape(b2, (1, D)), batch3, cnts, wc_pad,
      bc_pad)

    return logits_pad


# speedup vs baseline: 1.0031x; 1.0031x over previous
import functools

import jax
import jax.numpy as jnp
from jax import lax
from jax.experimental import pallas as pl
from jax.experimental.pallas import tpu as pltpu
from jax.experimental.pallas import tpu_sc as plsc

N_NODES = 10000
D = 128
NG = 64
NO = 10
NC = 2
NS = 16
NW = NC * NS
CH = 128

R = 1000
GRID = N_NODES // R
N_PAD = 10240
DEG_PAD = 10240


def _sc_degree(edge_ref_arr, E):
    nrows = E // CH
    IS = 4
    mesh = plsc.VectorSubcoreMesh(core_axis_name="c", subcore_axis_name="s")

    @functools.partial(
        pl.kernel,
        out_type=jax.ShapeDtypeStruct((NC, DEG_PAD), jnp.float32),
        mesh=mesh,
        scratch_types=[
            pltpu.VMEM((640,), jnp.float32),
            pltpu.VMEM((CH,), jnp.float32),
            pltpu.VMEM((IS, CH), jnp.int32),
            pltpu.VMEM_SHARED((DEG_PAD,), jnp.float32),
            pltpu.SemaphoreType.DMA((IS,)),
            pltpu.SemaphoreType.DMA((IS,)),
        ],
    )
    def k(e_ref, out_ref, zbuf, ones, didx, acc, jsem, ssem):
        c = lax.axis_index("c")
        s = lax.axis_index("s")
        wid = c * NS + s
        nch = (nrows - wid + NW - 1) // NW

        for p in range(2):
            dbase = pl.multiple_of((wid + p * NW) * CH, 8)
            pltpu.async_copy(e_ref.at[1, pl.ds(dbase, CH)], didx.at[p],
                             jsem.at[p])

        def zfill(i, carry):
            zbuf[pl.ds(i * 16, 16)] = jnp.zeros((16,), jnp.float32)
            return carry

        lax.fori_loop(0, 40, zfill, 0)
        for i in range(CH // 16):
            ones[pl.ds(i * 16, 16)] = jnp.ones((16,), jnp.float32)
        pltpu.sync_copy(zbuf, acc.at[pl.ds(s * 640, 640)])
        plsc.subcore_barrier()

        def body(j, carry):
            b = j % IS
            mb = pl.multiple_of((wid + j * NW) * CH, 8)
            pltpu.make_async_copy(e_ref.at[1, pl.ds(mb, CH)], didx.at[b],
                                  jsem.at[b]).wait()
            pltpu.async_copy(ones, acc.at[didx.at[b]], ssem.at[b], add=True)

            @pl.when(j + 2 < nch)
            def _():
                bn = (j + 2) % IS

                @pl.when(j >= 2)
                def _():
                    bo = (j - 2) % IS
                    pltpu.make_async_copy(ones, acc.at[didx.at[bo]],
                                          ssem.at[bo]).wait()

                mb2 = pl.multiple_of((wid + (j + 2) * NW) * CH, 8)
                pltpu.async_copy(e_ref.at[1, pl.ds(mb2, CH)], didx.at[bn],
                                 jsem.at[bn])

            return carry

        lax.fori_loop(0, nch, body, 0)
        for dj in range(4):
            jj = nch - 4 + dj

            @pl.when(jj >= 0)
            def _():
                b = jj % IS
                pltpu.make_async_copy(ones, acc.at[didx.at[b]],
                                      ssem.at[b]).wait()

        plsc.subcore_barrier()
        pltpu.sync_copy(acc.at[pl.ds(s * 640, 640)],
                        out_ref.at[c, pl.ds(s * 640, 640)])

    return k(edge_ref_arr)


def _sc_aggregate(g, edge_ref_arr, E):
    nrows = E // CH
    rpt = N_PAD // NS
    RS = 2
    IS = 6
    mesh = plsc.VectorSubcoreMesh(core_axis_name="c", subcore_axis_name="s")

    @functools.partial(
        pl.kernel,
        out_type=jax.ShapeDtypeStruct((NC, N_PAD, D), jnp.float32),
        mesh=mesh,
        scratch_types=[
            pltpu.VMEM((RS, CH, D), jnp.float32),
            pltpu.VMEM((IS, CH), jnp.int32),
            pltpu.VMEM((IS, CH), jnp.int32),
            pltpu.VMEM_SHARED((N_PAD, D), jnp.float32),
            pltpu.SemaphoreType.DMA((IS,)),
            pltpu.SemaphoreType.DMA((IS,)),
            pltpu.SemaphoreType.DMA((RS,)),
            pltpu.SemaphoreType.DMA((RS,)),
        ],
    )
    def k(g_ref, e_ref, out_ref, rows, sidx, didx, acc,
          isem, jsem, gsem, ssem):
        c = lax.axis_index("c")
        s = lax.axis_index("s")
        wid = c * NS + s
        nch = (nrows - wid + NW - 1) // NW

        for p in range(2):
            sb = pl.multiple_of((wid + p * NW) * CH, 8)
            pltpu.async_copy(e_ref.at[0, pl.ds(sb, CH)], sidx.at[p],
                             isem.at[p])
            pltpu.async_copy(e_ref.at[1, pl.ds(sb, CH)], didx.at[p],
                             jsem.at[p])

        def zrow(i, carry):
            for k16 in range(D // 16):
                rows[RS - 1, i, pl.ds(k16 * 16, 16)] = jnp.zeros(
                    (16,), jnp.float32)
            return carry

        lax.fori_loop(0, CH, zrow, 0)
        rbase = s * rpt
        for t in range(rpt // CH):
            pltpu.sync_copy(rows.at[RS - 1], acc.at[pl.ds(rbase + t * CH, CH)])
        sb0 = pl.multiple_of(wid * CH, 8)
        pltpu.make_async_copy(e_ref.at[0, pl.ds(sb0, CH)], sidx.at[0],
                              isem.at[0]).wait()
        pltpu.async_copy(g_ref.at[sidx.at[0]], rows.at[0], gsem.at[0])
        plsc.subcore_barrier()

        def body(j, carry):
            b = j % RS
            ib = j % IS

            @pl.when(j + 1 < nch)
            def _():
                bn = (j + 1) % RS
                ibn = (j + 1) % IS

                @pl.when(j + 1 - RS >= 0)
                def _():
                    jo = j + 1 - RS
                    pltpu.make_async_copy(rows.at[bn],
                                          acc.at[didx.at[jo % IS]],
                                          ssem.at[bn]).wait()

                sb1 = pl.multiple_of((wid + (j + 1) * NW) * CH, 8)
                pltpu.make_async_copy(e_ref.at[0, pl.ds(sb1, CH)],
                                      sidx.at[ibn], isem.at[ibn]).wait()
                pltpu.async_copy(g_ref.at[sidx.at[ibn]], rows.at[bn],
                                 gsem.at[bn])

            pltpu.make_async_copy(g_ref.at[sidx.at[ib]], rows.at[b],
                                  gsem.at[b]).wait()
            db0 = pl.multiple_of((wid + j * NW) * CH, 8)
            pltpu.make_async_copy(e_ref.at[1, pl.ds(db0, CH)], didx.at[ib],
                                  jsem.at[ib]).wait()
            pltpu.async_copy(rows.at[b], acc.at[didx.at[ib]], ssem.at[b],
                             add=True)

            @pl.when(j + 2 < nch)
            def _():
                ib2 = (j + 2) % IS
                sb2 = pl.multiple_of((wid + (j + 2) * NW) * CH, 8)
                pltpu.async_copy(e_ref.at[0, pl.ds(sb2, CH)], sidx.at[ib2],
                                 isem.at[ib2])
                pltpu.async_copy(e_ref.at[1, pl.ds(sb2, CH)], didx.at[ib2],
                                 jsem.at[ib2])

            return carry

        lax.fori_loop(0, nch, body, 0)
        for dj in range(RS):
            jj = nch - RS + dj

            @pl.when(jj >= 0)
            def _():
                pltpu.make_async_copy(rows.at[jj % RS],
                                      acc.at[didx.at[jj % IS]],
                                      ssem.at[jj % RS]).wait()

        plsc.subcore_barrier()
        pltpu.sync_copy(acc.at[pl.ds(rbase, rpt)],
                        out_ref.at[c, pl.ds(rbase, rpt)])

    return k(g, edge_ref_arr)


def _dinv_block(d_ref):
    deg = d_ref[...].astype(jnp.float32)
    return lax.rsqrt(jnp.maximum(deg, 1.0))


def _tc1_body(x_ref, w_ref, d_ref, g_ref):
    dinv = _dinv_block(d_ref)
    h = jnp.dot(x_ref[...].astype(jnp.bfloat16), w_ref[...],
                preferred_element_type=jnp.float32)
    g_ref[...] = h * dinv


def _tc2_body(a_ref0, a_ref1, g1_ref, d_ref, w_ref, b_ref, g2_ref):
    dinv = _dinv_block(d_ref)
    agg = (a_ref0[0] + a_ref1[0] + g1_ref[...]).astype(jnp.float32)
    x2 = jnp.maximum(dinv * agg + b_ref[...], 0.0)
    h = jnp.dot(x2.astype(jnp.bfloat16), w_ref[...],
                preferred_element_type=jnp.float32)
    g2_ref[...] = h * dinv


def _tc3_body(a_ref0, a_ref1, g2_ref, d_ref, b_ref, batch_ref,
              wc_ref, bc_ref, out_ref, sums, cnts):
    i = pl.program_id(0)
    dinv = _dinv_block(d_ref)
    agg = (a_ref0[0] + a_ref1[0] + g2_ref[...]).astype(jnp.float32)
    h3 = jnp.maximum(dinv * agg + b_ref[...], 0.0)
    bb = batch_ref[0, 0, :]
    onehot = jnp.equal(
        jnp.reshape(bb, (R, 1)),
        lax.broadcasted_iota(jnp.int32, (R, NG), 1)).astype(jnp.float32)
    ps = lax.dot_general(onehot.astype(jnp.bfloat16),
                         h3.astype(jnp.bfloat16), (((0,), (0,)), ((), ())),
                         preferred_element_type=jnp.float32)
    pc = lax.dot_general(onehot, jnp.ones((R, D), jnp.float32),
                         (((0,), (0,)), ((), ())),
                         preferred_element_type=jnp.float32)

    @pl.when(i == 0)
    def _():
        sums[...] = ps
        cnts[...] = pc

    @pl.when(i > 0)
    def _():
        sums[...] += ps
        cnts[...] += pc

    @pl.when(i == GRID - 1)
    def _():
        pooled = sums[...] / jnp.maximum(cnts[...], 1.0)
        logits = jnp.dot(pooled, wc_ref[...],
                         preferred_element_type=jnp.float32) + bc_ref[...]
        m = jnp.max(logits, axis=1, keepdims=True)
        sh = logits - m
        lse = jnp.log(jnp.sum(jnp.exp(sh), axis=1, keepdims=True))
        out_ref[...] = (sh - lse)[:, :NO]


def kernel(x, edge_index, batch, W1, b1, W2, b2, Wc, bc):
    E = edge_index.shape[1]

    deg = _sc_degree(edge_index, E)
    dsum = jnp.reshape((deg[0] + deg[1]).astype(jnp.bfloat16), (DEG_PAD, 1))

    row_spec = pl.BlockSpec((R, D), lambda i: (i, 0))
    aspec0 = pl.BlockSpec((1, R, D), lambda i: (0, i, 0))
    aspec1 = pl.BlockSpec((1, R, D), lambda i: (1, i, 0))
    dspec = pl.BlockSpec((R, 1), lambda i: (i, 0))
    wspec = pl.BlockSpec((D, D), lambda i: (0, 0))
    bspec = pl.BlockSpec((1, D), lambda i: (0, 0))

    g1 = pl.pallas_call(
        _tc1_body,
        grid=(GRID,),
        in_specs=[row_spec, wspec, dspec],
        out_specs=row_spec,
        out_shape=jax.ShapeDtypeStruct((N_NODES, D), jnp.float32),
    )(x, W1.astype(jnp.bfloat16), dsum)

    a1 = _sc_aggregate(g1, edge_index, E)

    g2 = pl.pallas_call(
        _tc2_body,
        grid=(GRID,),
        in_specs=[aspec0, aspec1, row_spec, dspec, wspec, bspec],
        out_specs=row_spec,
        out_shape=jax.ShapeDtypeStruct((N_NODES, D), jnp.float32),
    )(a1, a1, g1, dsum, W2.astype(jnp.bfloat16), jnp.reshape(b1, (1, D)))

    a2 = _sc_aggregate(g2, edge_index, E)

    batch3 = jnp.reshape(batch, (GRID, 1, R))
    wc_pad = jnp.zeros((D, D), jnp.float32).at[:, :Wc.shape[1]].set(Wc)
    bc_pad = jnp.full((1, D), -1e30, jnp.float32).at[0, :bc.shape[0]].set(bc)

    logits_pad = pl.pallas_call(
        _tc3_body,
        grid=(GRID,),
        in_specs=[aspec0, aspec1, row_spec, dspec, bspec,
                  pl.BlockSpec((1, 1, R), lambda i: (i, 0, 0)),
                  wspec, bspec],
        out_specs=pl.BlockSpec((NG, NO), lambda i: (0, 0)),
        out_shape=jax.ShapeDtypeStruct((NG, NO), jnp.float32),
        scratch_shapes=[pltpu.VMEM((NG, D), jnp.float32),
                        pltpu.VMEM((NG, D), jnp.float32)],
    )(a2, a2, g2, dsum, jnp.reshape(b2, (1, D)), batch3, wc_pad, bc_pad)

    return logits_pad
